# baseline (speedup 1.0000x reference)
import jax
import jax.numpy as jnp
from jax.experimental import pallas as pl
from jax.experimental.pallas import tpu as pltpu


def _make_tc_body(B, D, n_chunks):
    crows = B // n_chunks

    def tc_body(gu_hbm, gi_hbm, xui_hbm, gu_out_hbm, gi_out_hbm,
                bu, bv, bx, sem_u, sem_v, sem_x, sem_o):
        h_u, h_v = [], []
        for k in range(n_chunks):
            h_u.append(
                pltpu.async_copy(gu_hbm.at[pl.ds(k * crows, crows)],
                                 bu.at[k], sem_u.at[k]))
            h_v.append(
                pltpu.async_copy(gi_hbm.at[pl.ds(k * crows, crows)],
                                 bv.at[k], sem_v.at[k]))
        h_out = []
        for k in range(n_chunks):
            h_u[k].wait()
            h_v[k].wait()
            sl = pl.ds(k * crows, crows)
            h_out.append(
                pltpu.async_copy(bu.at[k], gu_out_hbm.at[sl], sem_o.at[k], priority=1))
            h_out.append(
                pltpu.async_copy(bv.at[k], gi_out_hbm.at[sl], sem_o.at[k], priority=1))
            bx[k] = jnp.sum(bu[k] * bv[k], axis=1)
            h_out.append(
                pltpu.async_copy(bx.at[k], xui_hbm.at[sl], sem_x.at[k]))
        for h in h_out:
            h.wait()

    return tc_body


def kernel(gu, gi):
    B, D = gu.shape
    n_chunks = 8
    crows = B // n_chunks
    gu = pltpu.with_memory_space_constraint(gu, pltpu.MemorySpace.HBM)
    gi = pltpu.with_memory_space_constraint(gi, pltpu.MemorySpace.HBM)
    xui, gu_o, gi_o = pl.pallas_call(
        _make_tc_body(B, D, n_chunks),
        in_specs=[
            pl.BlockSpec(memory_space=pltpu.MemorySpace.HBM),
            pl.BlockSpec(memory_space=pltpu.MemorySpace.HBM),
        ],
        out_specs=[
            pl.BlockSpec(memory_space=pltpu.MemorySpace.HBM),
            pl.BlockSpec(memory_space=pltpu.MemorySpace.HBM),
            pl.BlockSpec(memory_space=pltpu.MemorySpace.HBM),
        ],
        out_shape=[
            jax.ShapeDtypeStruct((B,), jnp.float32),
            jax.ShapeDtypeStruct((B, D), jnp.float32),
            jax.ShapeDtypeStruct((B, D), jnp.float32),
        ],
        scratch_shapes=[
            pltpu.VMEM((n_chunks, crows, D), jnp.float32),
            pltpu.VMEM((n_chunks, crows, D), jnp.float32),
            pltpu.VMEM((n_chunks, crows), jnp.float32),
            pltpu.SemaphoreType.DMA((n_chunks,)),
            pltpu.SemaphoreType.DMA((n_chunks,)),
            pltpu.SemaphoreType.DMA((n_chunks,)),
            pltpu.SemaphoreType.DMA((n_chunks,)),
        ],
    )(gu, gi)
    return (xui, gu_o, gi_o)


# speedup vs baseline: 1.0123x; 1.0123x over previous
import jax
import jax.numpy as jnp
from jax.experimental import pallas as pl
from jax.experimental.pallas import tpu as pltpu


def _make_tc_body(B, D, n_chunks):
    crows = B // n_chunks

    def tc_body(gu_hbm, gi_hbm, xui_hbm, gu_out_hbm, gi_out_hbm,
                bu, bv, bx, sem_u, sem_v, sem_x, sem_o):
        h_u, h_v = [], []
        for k in range(n_chunks):
            h_u.append(
                pltpu.async_copy(gu_hbm.at[pl.ds(k * crows, crows)],
                                 bu.at[k], sem_u.at[k]))
            h_v.append(
                pltpu.async_copy(gi_hbm.at[pl.ds(k * crows, crows)],
                                 bv.at[k], sem_v.at[k]))
        h_out = []
        for k in range(n_chunks):
            h_u[k].wait()
            h_v[k].wait()
            sl = pl.ds(k * crows, crows)
            h_out.append(
                pltpu.async_copy(bu.at[k], gu_out_hbm.at[sl], sem_o.at[k]))
            h_out.append(
                pltpu.async_copy(bv.at[k], gi_out_hbm.at[sl], sem_o.at[k]))
            bx[k] = jnp.sum(bu[k] * bv[k], axis=1)
            h_out.append(
                pltpu.async_copy(bx.at[k], xui_hbm.at[sl], sem_x.at[k]))
        for h in h_out:
            h.wait()

    return tc_body


def kernel(gu, gi):
    B, D = gu.shape
    n_chunks = 8
    crows = B // n_chunks
    gu = pltpu.with_memory_space_constraint(gu, pltpu.MemorySpace.HBM)
    gi = pltpu.with_memory_space_constraint(gi, pltpu.MemorySpace.HBM)
    xui, gu_o, gi_o = pl.pallas_call(
        _make_tc_body(B, D, n_chunks),
        in_specs=[
            pl.BlockSpec(memory_space=pltpu.MemorySpace.HBM),
            pl.BlockSpec(memory_space=pltpu.MemorySpace.HBM),
        ],
        out_specs=[
            pl.BlockSpec(memory_space=pltpu.MemorySpace.HBM),
            pl.BlockSpec(memory_space=pltpu.MemorySpace.HBM),
            pl.BlockSpec(memory_space=pltpu.MemorySpace.HBM),
        ],
        out_shape=[
            jax.ShapeDtypeStruct((B,), jnp.float32),
            jax.ShapeDtypeStruct((B, D), jnp.float32),
            jax.ShapeDtypeStruct((B, D), jnp.float32),
        ],
        scratch_shapes=[
            pltpu.VMEM((n_chunks, crows, D), jnp.float32),
            pltpu.VMEM((n_chunks, crows, D), jnp.float32),
            pltpu.VMEM((n_chunks, crows), jnp.float32),
            pltpu.SemaphoreType.DMA((n_chunks,)),
            pltpu.SemaphoreType.DMA((n_chunks,)),
            pltpu.SemaphoreType.DMA((n_chunks,)),
            pltpu.SemaphoreType.DMA((n_chunks,)),
        ],
    )(gu, gi)
    return (xui, gu_o, gi_o)
